# baseline (speedup 1.0000x reference)
import functools

import jax
import jax.numpy as jnp
from jax import lax
from jax.experimental import pallas as pl
from jax.experimental.pallas import tpu as pltpu
from jax.experimental.pallas import tpu_sc as plsc

N = 10000
D = 128
E = 320000

NC = 2
NS = 16
NW = NC * NS
EW = E // NW
B = 80
NB = EW // B
NBUF = 5
NG = NB // NBUF
BA = 40
NBA = EW // BA
NGA = NBA // NBUF
NP = 10240
SEG = NP // NS
RPT = NP // NS
ZCH = 16

_mesh = plsc.VectorSubcoreMesh(
    core_axis_name="c", subcore_axis_name="s", num_cores=NC, num_subcores=NS)

_f32 = jnp.float32


@functools.partial(
    pl.kernel,
    out_type=jax.ShapeDtypeStruct((NC, NP), _f32),
    mesh=_mesh,
    scratch_types=[
        pltpu.VMEM_SHARED((NP,), _f32),
        pltpu.VMEM((NB, B), jnp.int32),
        pltpu.VMEM((B,), _f32),
        pltpu.VMEM((SEG,), _f32),
        pltpu.SemaphoreType.DMA((NBUF,)),
    ],
)
def _sc_deg(dst_hbm, out_hbm, dacc, didx, ones_v, zbuf, sem):
    c = lax.axis_index("c")
    s = lax.axis_index("s")
    w = c * NS + s

    z16 = jnp.zeros((16,), _f32)
    for j in range(SEG // 16):
        zbuf[pl.ds(j * 16, 16)] = z16
    for j in range(B // 16):
        ones_v[pl.ds(j * 16, 16)] = jnp.full((16,), 1.0, _f32)

    pltpu.sync_copy(zbuf, dacc.at[pl.ds(s * SEG, SEG)])
    pltpu.sync_copy(dst_hbm.at[w], didx)
    plsc.subcore_barrier()

    def _scat(i, b):
        return pltpu.make_async_copy(ones_v, dacc.at[didx.at[i]], sem.at[b])

    @pl.loop(0, NG)
    def _deg_loop(g):
        for b in range(NBUF):
            i = g * NBUF + b

            @pl.when(g > 0)
            def _():
                _scat(i, b).wait()

            _scat(i, b).start(add=True)

    for b in range(NBUF):
        _scat((NG - 1) * NBUF + b, b).wait()

    plsc.subcore_barrier()
    pltpu.sync_copy(dacc.at[pl.ds(s * SEG, SEG)],
                    out_hbm.at[c, pl.ds(s * SEG, SEG)])


@functools.partial(
    pl.kernel,
    out_type=jax.ShapeDtypeStruct((NC, NP, D), _f32),
    mesh=_mesh,
    scratch_types=[
        pltpu.VMEM_SHARED((NP, D), _f32),
        pltpu.VMEM((NBUF, BA, D), _f32),
        pltpu.VMEM((ZCH, D), _f32),
        [pltpu.VMEM((BA,), jnp.int32) for _ in range(NBUF)],
        [pltpu.VMEM((BA,), jnp.int32) for _ in range(NBUF)],
        pltpu.SemaphoreType.DMA((NBUF,)),
        pltpu.SemaphoreType.DMA((NBUF,)),
        pltpu.SemaphoreType.DMA((NBUF,)),
    ],
)
def _sc_agg(y_hbm, src_hbm, dst_hbm, out_hbm, acc, rows, zbuf, sidx, didx,
            sem_i, sem_g, sem_s):
    c = lax.axis_index("c")
    s = lax.axis_index("s")
    w = c * NS + s

    z16 = jnp.zeros((16,), _f32)
    for r in range(ZCH):
        for j in range(D // 16):
            zbuf[r, pl.ds(j * 16, 16)] = z16

    def _ld_idx(i, b):
        base = w * EW + i * BA
        return (pltpu.make_async_copy(src_hbm.at[pl.ds(base, BA)], sidx[b],
                                      sem_i.at[b]),
                pltpu.make_async_copy(dst_hbm.at[pl.ds(base, BA)], didx[b],
                                      sem_i.at[b]))

    def _gath(b):
        return pltpu.make_async_copy(y_hbm.at[sidx[b]], rows.at[b],
                                     sem_g.at[b])

    def _scat(b):
        return pltpu.make_async_copy(rows.at[b], acc.at[didx[b]],
                                     sem_s.at[b])

    for b in range(NBUF):
        for d in _ld_idx(b, b):
            d.start()

    @pl.when(c == 0)
    def _():

        @pl.when(s < NS - 1)
        def _():
            pltpu.sync_copy(y_hbm.at[pl.ds(s * RPT, RPT)],
                            acc.at[pl.ds(s * RPT, RPT)])

        @pl.when(s == NS - 1)
        def _():
            pltpu.sync_copy(y_hbm.at[pl.ds((NS - 1) * RPT, N - (NS - 1) * RPT)],
                            acc.at[pl.ds((NS - 1) * RPT, N - (NS - 1) * RPT)])

            @pl.loop(0, (NP - N) // ZCH)
            def _zpad(k):
                pltpu.sync_copy(zbuf, acc.at[pl.ds(N + k * ZCH, ZCH)])

    @pl.when(c == 1)
    def _():

        @pl.loop(0, RPT // ZCH)
        def _zero_loop(k):
            pltpu.sync_copy(zbuf, acc.at[pl.ds(s * RPT + k * ZCH, ZCH)])

    plsc.subcore_barrier()

    for b in range(NBUF):
        for d in _ld_idx(b, b):
            d.wait()
        _gath(b).start()

    @pl.loop(0, NGA)
    def _edge_loop(g):
        for b in range(NBUF):
            _gath(b).wait()
            _scat(b).start(add=True)
        for b in range(NBUF):

            @pl.when(g < NGA - 1)
            def _():
                _scat(b).wait()
                nxt = (g + 1) * NBUF + b
                dsc = _ld_idx(nxt, b)
                for d in dsc:
                    d.start()
                for d in dsc:
                    d.wait()
                _gath(b).start()

    for b in range(NBUF):
        _scat(b).wait()

    plsc.subcore_barrier()
    pltpu.sync_copy(acc.at[pl.ds(s * RPT, RPT)],
                    out_hbm.at[c, pl.ds(s * RPT, RPT)])


def _dis(d0, d1):
    return lax.rsqrt(d0 + d1 + 1.0)


def _leaky(v):
    return jnp.where(v >= 0, v, 0.01 * v)


def _layer_norm(v, g, b):
    mu = jnp.mean(v, axis=-1, keepdims=True)
    var = jnp.mean((v - mu) ** 2, axis=-1, keepdims=True)
    return (v - mu) * lax.rsqrt(var + 1e-5) * g + b


def _mm_body(x_ref, w_ref, o_ref):
    o_ref[...] = jnp.dot(x_ref[...], w_ref[...], preferred_element_type=_f32)


def _scale_body(d0_ref, d1_ref, xw_ref, y_ref, dis_ref):
    dis = _dis(d0_ref[...], d1_ref[...])
    dis_ref[...] = dis
    y_ref[...] = dis * xw_ref[...]


def _tc2_body(p_ref, dis_ref, x_ref, w_ref,
              b1_ref, g1_ref, bt1_ref, y2_ref):
    dis = dis_ref[...]
    h = dis * (p_ref[0, :N] + p_ref[1, :N]) + b1_ref[...]
    h = _layer_norm(_leaky(h) + x_ref[...], g1_ref[...], bt1_ref[...])
    y2_ref[...] = dis * jnp.dot(h, w_ref[...], preferred_element_type=_f32)


def _tc3_body(p_ref, dis_ref, x_ref,
              b2_ref, g2_ref, bt2_ref, out_ref):
    dis = dis_ref[...]
    h2 = dis * (p_ref[0, :N] + p_ref[1, :N]) + b2_ref[...]
    out_ref[...] = _leaky(
        _layer_norm(h2 + x_ref[...], g2_ref[...], bt2_ref[...]))


def _tc(body, *args, out_shape=None):
    if out_shape is None:
        out_shape = jax.ShapeDtypeStruct((N, D), _f32)
    return pl.pallas_call(body, out_shape=out_shape)(*args)


def kernel(x, edge_index, W1, b1, g1, bt1, W2, b2, g2, bt2):
    src = edge_index[0]
    dst = edge_index[1]
    dpc = _sc_deg(dst.reshape(NW, NB, B))
    d0 = dpc[0, :N].reshape(N, 1)
    d1 = dpc[1, :N].reshape(N, 1)

    xw1 = _tc(_mm_body, x, W1)
    y1, dis = _tc(_scale_body, d0, d1, xw1,
                  out_shape=(jax.ShapeDtypeStruct((N, D), _f32),
                             jax.ShapeDtypeStruct((N, 1), _f32)))
    p1 = _sc_agg(y1, src, dst)
    y2 = _tc(_tc2_body, p1, dis, x, W2,
             b1.reshape(1, D), g1.reshape(1, D), bt1.reshape(1, D))
    p2 = _sc_agg(y2, src, dst)
    return _tc(_tc3_body, p2, dis, x,
               b2.reshape(1, D), g2.reshape(1, D), bt2.reshape(1, D))


# speedup vs baseline: 1.0228x; 1.0228x over previous
import functools

import jax
import jax.numpy as jnp
from jax import lax
from jax.experimental import pallas as pl
from jax.experimental.pallas import tpu as pltpu
from jax.experimental.pallas import tpu_sc as plsc

N = 10000
D = 128
E = 320000

NC = 2
NS = 16
NW = NC * NS
EW = E // NW
B = 80
NB = EW // B
NBUF = 5
NG = NB // NBUF
BA = 40
NBA = EW // BA
NGA = NBA // NBUF
NP = 10240
SEG = NP // NS
RPT = NP // NS
ZCH = 16

_mesh = plsc.VectorSubcoreMesh(
    core_axis_name="c", subcore_axis_name="s", num_cores=NC, num_subcores=NS)

_f32 = jnp.float32


@functools.partial(
    pl.kernel,
    out_type=jax.ShapeDtypeStruct((NC, NP), _f32),
    mesh=_mesh,
    scratch_types=[
        pltpu.VMEM_SHARED((NP,), _f32),
        pltpu.VMEM((NB, B), jnp.int32),
        pltpu.VMEM((B,), _f32),
        pltpu.VMEM((SEG,), _f32),
        pltpu.SemaphoreType.DMA((NBUF,)),
    ],
)
def _sc_deg(dst_hbm, out_hbm, dacc, didx, ones_v, zbuf, sem):
    c = lax.axis_index("c")
    s = lax.axis_index("s")
    w = c * NS + s

    z16 = jnp.zeros((16,), _f32)
    for j in range(SEG // 16):
        zbuf[pl.ds(j * 16, 16)] = z16
    for j in range(B // 16):
        ones_v[pl.ds(j * 16, 16)] = jnp.full((16,), 1.0, _f32)

    pltpu.sync_copy(zbuf, dacc.at[pl.ds(s * SEG, SEG)])
    pltpu.sync_copy(dst_hbm.at[w], didx)
    plsc.subcore_barrier()

    def _scat(i, b):
        return pltpu.make_async_copy(ones_v, dacc.at[didx.at[i]], sem.at[b])

    @pl.loop(0, NG)
    def _deg_loop(g):
        for b in range(NBUF):
            i = g * NBUF + b

            @pl.when(g > 0)
            def _():
                _scat(i, b).wait()

            _scat(i, b).start(add=True)

    for b in range(NBUF):
        _scat((NG - 1) * NBUF + b, b).wait()

    plsc.subcore_barrier()
    pltpu.sync_copy(dacc.at[pl.ds(s * SEG, SEG)],
                    out_hbm.at[c, pl.ds(s * SEG, SEG)])


@functools.partial(
    pl.kernel,
    out_type=jax.ShapeDtypeStruct((NC, NP, D), _f32),
    mesh=_mesh,
    scratch_types=[
        pltpu.VMEM_SHARED((NP, D), _f32),
        pltpu.VMEM((NBUF, BA, D), _f32),
        pltpu.VMEM((ZCH, D), _f32),
        [pltpu.VMEM((BA,), jnp.int32) for _ in range(NBUF)],
        [pltpu.VMEM((BA,), jnp.int32) for _ in range(NBUF)],
        pltpu.SemaphoreType.DMA((NBUF,)),
        pltpu.SemaphoreType.DMA((NBUF,)),
        pltpu.SemaphoreType.DMA((NBUF,)),
    ],
)
def _sc_agg(y_hbm, src_hbm, dst_hbm, out_hbm, acc, rows, zbuf, sidx, didx,
            sem_i, sem_g, sem_s):
    c = lax.axis_index("c")
    s = lax.axis_index("s")
    w = c * NS + s

    z16 = jnp.zeros((16,), _f32)
    for r in range(ZCH):
        for j in range(D // 16):
            zbuf[r, pl.ds(j * 16, 16)] = z16

    def _ld_idx(i, b):
        base = w * EW + i * BA
        return (pltpu.make_async_copy(src_hbm.at[pl.ds(base, BA)], sidx[b],
                                      sem_i.at[b]),
                pltpu.make_async_copy(dst_hbm.at[pl.ds(base, BA)], didx[b],
                                      sem_i.at[b]))

    def _gath(b):
        return pltpu.make_async_copy(y_hbm.at[sidx[b]], rows.at[b],
                                     sem_g.at[b])

    def _scat(b):
        return pltpu.make_async_copy(rows.at[b], acc.at[didx[b]],
                                     sem_s.at[b])

    for b in range(NBUF):
        for d in _ld_idx(b, b):
            d.start()

    @pl.when(c == 0)
    def _():

        @pl.when(s < NS - 1)
        def _():
            pltpu.sync_copy(y_hbm.at[pl.ds(s * RPT, RPT)],
                            acc.at[pl.ds(s * RPT, RPT)])

        @pl.when(s == NS - 1)
        def _():
            pltpu.sync_copy(y_hbm.at[pl.ds((NS - 1) * RPT, N - (NS - 1) * RPT)],
                            acc.at[pl.ds((NS - 1) * RPT, N - (NS - 1) * RPT)])

            @pl.loop(0, (NP - N) // ZCH)
            def _zpad(k):
                pltpu.sync_copy(zbuf, acc.at[pl.ds(N + k * ZCH, ZCH)])

    @pl.when(c == 1)
    def _():

        @pl.loop(0, RPT // ZCH)
        def _zero_loop(k):
            pltpu.sync_copy(zbuf, acc.at[pl.ds(s * RPT + k * ZCH, ZCH)])

    plsc.subcore_barrier()

    for b in range(NBUF):
        for d in _ld_idx(b, b):
            d.wait()
        _gath(b).start()

    @pl.loop(0, NGA)
    def _edge_loop(g):
        for b in range(NBUF):
            _gath(b).wait()
            _scat(b).start(add=True)
        for b in range(NBUF):

            @pl.when(g < NGA - 1)
            def _():
                _scat(b).wait()
                nxt = (g + 1) * NBUF + b
                dsc = _ld_idx(nxt, b)
                for d in dsc:
                    d.start()
                for d in dsc:
                    d.wait()
                _gath(b).start()

    for b in range(NBUF):
        _scat(b).wait()

    plsc.subcore_barrier()
    pltpu.sync_copy(acc.at[pl.ds(s * RPT, RPT)],
                    out_hbm.at[c, pl.ds(s * RPT, RPT)])


def _dis(d0, d1):
    return lax.rsqrt(d0 + d1 + 1.0)


def _leaky(v):
    return jnp.where(v >= 0, v, 0.01 * v)


def _layer_norm(v, g, b):
    mu = jnp.mean(v, axis=-1, keepdims=True)
    var = jnp.mean((v - mu) ** 2, axis=-1, keepdims=True)
    return (v - mu) * lax.rsqrt(var + 1e-5) * g + b


def _mm_body(x_ref, w_ref, o_ref):
    o_ref[...] = jnp.dot(x_ref[...], w_ref[...], preferred_element_type=_f32)


def _scale_body(dpc_ref, xw_ref, y_ref, dis_ref):
    deg_row = dpc_ref[0:1, :] + dpc_ref[1:2, :]
    deg_col = jnp.transpose(deg_row)[:N, :]
    dis = lax.rsqrt(deg_col + 1.0)
    dis_ref[...] = dis
    y_ref[...] = dis * xw_ref[...]


def _tc2_body(p_ref, dis_ref, x_ref, w_ref,
              b1_ref, g1_ref, bt1_ref, y2_ref):
    dis = dis_ref[...]
    h = dis * (p_ref[0, :N] + p_ref[1, :N]) + b1_ref[...]
    h = _layer_norm(_leaky(h) + x_ref[...], g1_ref[...], bt1_ref[...])
    y2_ref[...] = dis * jnp.dot(h, w_ref[...], preferred_element_type=_f32)


def _tc3_body(p_ref, dis_ref, x_ref,
              b2_ref, g2_ref, bt2_ref, out_ref):
    dis = dis_ref[...]
    h2 = dis * (p_ref[0, :N] + p_ref[1, :N]) + b2_ref[...]
    out_ref[...] = _leaky(
        _layer_norm(h2 + x_ref[...], g2_ref[...], bt2_ref[...]))


def _tc(body, *args, out_shape=None):
    if out_shape is None:
        out_shape = jax.ShapeDtypeStruct((N, D), _f32)
    return pl.pallas_call(body, out_shape=out_shape)(*args)


def kernel(x, edge_index, W1, b1, g1, bt1, W2, b2, g2, bt2):
    src = edge_index[0]
    dst = edge_index[1]
    dpc = _sc_deg(dst.reshape(NW, NB, B))

    xw1 = _tc(_mm_body, x, W1)
    y1, dis = _tc(_scale_body, dpc, xw1,
                  out_shape=(jax.ShapeDtypeStruct((N, D), _f32),
                             jax.ShapeDtypeStruct((N, 1), _f32)))
    p1 = _sc_agg(y1, src, dst)
    y2 = _tc(_tc2_body, p1, dis, x, W2,
             b1.reshape(1, D), g1.reshape(1, D), bt1.reshape(1, D))
    p2 = _sc_agg(y2, src, dst)
    return _tc(_tc3_body, p2, dis, x,
               b2.reshape(1, D), g2.reshape(1, D), bt2.reshape(1, D))


# speedup vs baseline: 1.0235x; 1.0007x over previous
import functools

import jax
import jax.numpy as jnp
from jax import lax
from jax.experimental import pallas as pl
from jax.experimental.pallas import tpu as pltpu
from jax.experimental.pallas import tpu_sc as plsc

N = 10000
D = 128
E = 320000

NC = 2
NS = 16
NW = NC * NS
EW = E // NW
B = 80
NB = EW // B
NBUF = 5
NG = NB // NBUF
BA = 40
NBA = EW // BA
NGA = NBA // NBUF
NP = 10240
SEG = NP // NS
RPT = NP // NS
ZCH = 16

_mesh = plsc.VectorSubcoreMesh(
    core_axis_name="c", subcore_axis_name="s", num_cores=NC, num_subcores=NS)

_f32 = jnp.float32


@functools.partial(
    pl.kernel,
    out_type=jax.ShapeDtypeStruct((NC, NP), _f32),
    mesh=_mesh,
    scratch_types=[
        pltpu.VMEM_SHARED((NP,), _f32),
        pltpu.VMEM((NB, B), jnp.int32),
        pltpu.VMEM((B,), _f32),
        pltpu.VMEM((SEG,), _f32),
        pltpu.SemaphoreType.DMA((NBUF,)),
    ],
)
def _sc_deg(dst_hbm, out_hbm, dacc, didx, ones_v, zbuf, sem):
    c = lax.axis_index("c")
    s = lax.axis_index("s")
    w = c * NS + s

    z16 = jnp.zeros((16,), _f32)
    for j in range(SEG // 16):
        zbuf[pl.ds(j * 16, 16)] = z16
    for j in range(B // 16):
        ones_v[pl.ds(j * 16, 16)] = jnp.full((16,), 1.0, _f32)

    pltpu.sync_copy(zbuf, dacc.at[pl.ds(s * SEG, SEG)])
    pltpu.sync_copy(dst_hbm.at[w], didx)
    plsc.subcore_barrier()

    def _scat(i, b):
        return pltpu.make_async_copy(ones_v, dacc.at[didx.at[i]], sem.at[b])

    @pl.loop(0, NG)
    def _deg_loop(g):
        for b in range(NBUF):
            i = g * NBUF + b

            @pl.when(g > 0)
            def _():
                _scat(i, b).wait()

            _scat(i, b).start(add=True)

    for b in range(NBUF):
        _scat((NG - 1) * NBUF + b, b).wait()

    plsc.subcore_barrier()
    pltpu.sync_copy(dacc.at[pl.ds(s * SEG, SEG)],
                    out_hbm.at[c, pl.ds(s * SEG, SEG)])


@functools.partial(
    pl.kernel,
    out_type=jax.ShapeDtypeStruct((NC, NP, D), _f32),
    mesh=_mesh,
    scratch_types=[
        pltpu.VMEM_SHARED((NP, D), _f32),
        pltpu.VMEM((NBUF, BA, D), _f32),
        pltpu.VMEM((ZCH, D), _f32),
        [pltpu.VMEM((BA,), jnp.int32) for _ in range(NBUF)],
        [pltpu.VMEM((BA,), jnp.int32) for _ in range(NBUF)],
        pltpu.SemaphoreType.DMA((NBUF,)),
        pltpu.SemaphoreType.DMA((NBUF,)),
        pltpu.SemaphoreType.DMA((NBUF,)),
    ],
)
def _sc_agg(y_hbm, src_hbm, dst_hbm, out_hbm, acc, rows, zbuf, sidx, didx,
            sem_i, sem_g, sem_s):
    c = lax.axis_index("c")
    s = lax.axis_index("s")
    w = c * NS + s

    z16 = jnp.zeros((16,), _f32)
    for r in range(ZCH):
        for j in range(D // 16):
            zbuf[r, pl.ds(j * 16, 16)] = z16

    def _ld_idx(i, b):
        base = w * EW + i * BA
        return (pltpu.make_async_copy(src_hbm.at[pl.ds(base, BA)], sidx[b],
                                      sem_i.at[b]),
                pltpu.make_async_copy(dst_hbm.at[pl.ds(base, BA)], didx[b],
                                      sem_i.at[b]))

    def _gath(b):
        return pltpu.make_async_copy(y_hbm.at[sidx[b]], rows.at[b],
                                     sem_g.at[b])

    def _scat(b):
        return pltpu.make_async_copy(rows.at[b], acc.at[didx[b]],
                                     sem_s.at[b])

    for b in range(NBUF):
        for d in _ld_idx(b, b):
            d.start()

    @pl.when(c == 0)
    def _():

        @pl.when(s < NS - 1)
        def _():
            pltpu.sync_copy(y_hbm.at[pl.ds(s * RPT, RPT)],
                            acc.at[pl.ds(s * RPT, RPT)])

        @pl.when(s == NS - 1)
        def _():
            pltpu.sync_copy(y_hbm.at[pl.ds((NS - 1) * RPT, N - (NS - 1) * RPT)],
                            acc.at[pl.ds((NS - 1) * RPT, N - (NS - 1) * RPT)])

            @pl.loop(0, (NP - N) // ZCH)
            def _zpad(k):
                pltpu.sync_copy(zbuf, acc.at[pl.ds(N + k * ZCH, ZCH)])

    @pl.when(c == 1)
    def _():

        @pl.loop(0, RPT // ZCH)
        def _zero_loop(k):
            pltpu.sync_copy(zbuf, acc.at[pl.ds(s * RPT + k * ZCH, ZCH)])

    plsc.subcore_barrier()

    for b in range(NBUF):
        for d in _ld_idx(b, b):
            d.wait()
        _gath(b).start()

    @pl.loop(0, NGA)
    def _edge_loop(g):
        for b in range(NBUF):
            _gath(b).wait()
            _scat(b).start(add=True)
        for b in range(NBUF):

            @pl.when(g < NGA - 1)
            def _():
                _scat(b).wait()
                nxt = (g + 1) * NBUF + b
                dsc = _ld_idx(nxt, b)
                for d in dsc:
                    d.start()
                for d in dsc:
                    d.wait()
                _gath(b).start()

    for b in range(NBUF):
        _scat(b).wait()

    plsc.subcore_barrier()
    pltpu.sync_copy(acc.at[pl.ds(s * RPT, RPT)],
                    out_hbm.at[c, pl.ds(s * RPT, RPT)])


def _dis(d0, d1):
    return lax.rsqrt(d0 + d1 + 1.0)


def _leaky(v):
    return jnp.where(v >= 0, v, 0.01 * v)


def _layer_norm(v, g, b):
    mu = jnp.mean(v, axis=-1, keepdims=True)
    var = jnp.mean((v - mu) ** 2, axis=-1, keepdims=True)
    return (v - mu) * lax.rsqrt(var + 1e-5) * g + b


def _mm_body(x_ref, w_ref, o_ref):
    o_ref[...] = jnp.dot(x_ref[...], w_ref[...], preferred_element_type=_f32)


def _dis_col(dpc_ref):
    deg_row = dpc_ref[0:1, :] + dpc_ref[1:2, :]
    deg_col = jnp.transpose(deg_row)[:N, :]
    return lax.rsqrt(deg_col + 1.0)


def _scale_body(dpc_ref, xw_ref, y_ref):
    y_ref[...] = _dis_col(dpc_ref) * xw_ref[...]


def _tc2_body(p_ref, dpc_ref, x_ref, w_ref,
              b1_ref, g1_ref, bt1_ref, y2_ref):
    dis = _dis_col(dpc_ref)
    h = dis * (p_ref[0, :N] + p_ref[1, :N]) + b1_ref[...]
    h = _layer_norm(_leaky(h) + x_ref[...], g1_ref[...], bt1_ref[...])
    y2_ref[...] = dis * jnp.dot(h, w_ref[...], preferred_element_type=_f32)


def _tc3_body(p_ref, dpc_ref, x_ref,
              b2_ref, g2_ref, bt2_ref, out_ref):
    dis = _dis_col(dpc_ref)
    h2 = dis * (p_ref[0, :N] + p_ref[1, :N]) + b2_ref[...]
    out_ref[...] = _leaky(
        _layer_norm(h2 + x_ref[...], g2_ref[...], bt2_ref[...]))


def _tc(body, *args, out_shape=None):
    if out_shape is None:
        out_shape = jax.ShapeDtypeStruct((N, D), _f32)
    return pl.pallas_call(body, out_shape=out_shape)(*args)


def kernel(x, edge_index, W1, b1, g1, bt1, W2, b2, g2, bt2):
    src = edge_index[0]
    dst = edge_index[1]
    dpc = _sc_deg(dst.reshape(NW, NB, B))

    xw1 = _tc(_mm_body, x, W1)
    y1 = _tc(_scale_body, dpc, xw1)
    p1 = _sc_agg(y1, src, dst)
    y2 = _tc(_tc2_body, p1, dpc, x, W2,
             b1.reshape(1, D), g1.reshape(1, D), bt1.reshape(1, D))
    p2 = _sc_agg(y2, src, dst)
    return _tc(_tc3_body, p2, dpc, x,
               b2.reshape(1, D), g2.reshape(1, D), bt2.reshape(1, D))


# speedup vs baseline: 1.0236x; 1.0001x over previous
import functools

import jax
import jax.numpy as jnp
from jax import lax
from jax.experimental import pallas as pl
from jax.experimental.pallas import tpu as pltpu
from jax.experimental.pallas import tpu_sc as plsc

N = 10000
D = 128
E = 320000

NC = 2
NS = 16
NW = NC * NS
EW = E // NW
B = 80
NB = EW // B
NBUF = 5
NG = NB // NBUF
BA = 40
NBA = EW // BA
NGA = NBA // NBUF
NP = 10240
SEG = NP // NS
RPT = NP // NS
ZCH = 16

_mesh = plsc.VectorSubcoreMesh(
    core_axis_name="c", subcore_axis_name="s", num_cores=NC, num_subcores=NS)

_f32 = jnp.float32


@functools.partial(
    pl.kernel,
    out_type=jax.ShapeDtypeStruct((NC, NP), _f32),
    mesh=_mesh,
    scratch_types=[
        pltpu.VMEM_SHARED((NP,), _f32),
        pltpu.VMEM((NB, B), jnp.int32),
        pltpu.VMEM((B,), _f32),
        pltpu.VMEM((SEG,), _f32),
        pltpu.SemaphoreType.DMA((NBUF,)),
    ],
)
def _sc_deg(dst_hbm, out_hbm, dacc, didx, ones_v, zbuf, sem):
    c = lax.axis_index("c")
    s = lax.axis_index("s")
    w = c * NS + s

    z16 = jnp.zeros((16,), _f32)
    for j in range(SEG // 16):
        zbuf[pl.ds(j * 16, 16)] = z16
    for j in range(B // 16):
        ones_v[pl.ds(j * 16, 16)] = jnp.full((16,), 1.0, _f32)

    pltpu.sync_copy(zbuf, dacc.at[pl.ds(s * SEG, SEG)])
    pltpu.sync_copy(dst_hbm.at[w], didx)
    plsc.subcore_barrier()

    def _scat(i, b):
        return pltpu.make_async_copy(ones_v, dacc.at[didx.at[i]], sem.at[b])

    @pl.loop(0, NG)
    def _deg_loop(g):
        for b in range(NBUF):
            i = g * NBUF + b

            @pl.when(g > 0)
            def _():
                _scat(i, b).wait()

            _scat(i, b).start(add=True)

    for b in range(NBUF):
        _scat((NG - 1) * NBUF + b, b).wait()

    plsc.subcore_barrier()
    pltpu.sync_copy(dacc.at[pl.ds(s * SEG, SEG)],
                    out_hbm.at[c, pl.ds(s * SEG, SEG)])


@functools.partial(
    pl.kernel,
    out_type=jax.ShapeDtypeStruct((NC, NP, D), _f32),
    mesh=_mesh,
    scratch_types=[
        pltpu.VMEM_SHARED((NP, D), _f32),
        pltpu.VMEM((NBUF, BA, D), _f32),
        pltpu.VMEM((ZCH, D), _f32),
        [pltpu.VMEM((BA,), jnp.int32) for _ in range(NBUF)],
        [pltpu.VMEM((BA,), jnp.int32) for _ in range(NBUF)],
        pltpu.SemaphoreType.DMA((NBUF,)),
        pltpu.SemaphoreType.DMA((NBUF,)),
        pltpu.SemaphoreType.DMA((NBUF,)),
    ],
)
def _sc_agg(y_hbm, src_hbm, dst_hbm, out_hbm, acc, rows, zbuf, sidx, didx,
            sem_i, sem_g, sem_s):
    c = lax.axis_index("c")
    s = lax.axis_index("s")
    w = c * NS + s

    z16 = jnp.zeros((16,), _f32)
    for r in range(ZCH):
        for j in range(D // 16):
            zbuf[r, pl.ds(j * 16, 16)] = z16

    def _ld_idx(i, b):
        base = w * EW + i * BA
        return (pltpu.make_async_copy(src_hbm.at[pl.ds(base, BA)], sidx[b],
                                      sem_i.at[b]),
                pltpu.make_async_copy(dst_hbm.at[pl.ds(base, BA)], didx[b],
                                      sem_i.at[b]))

    def _gath(b):
        return pltpu.make_async_copy(y_hbm.at[sidx[b]], rows.at[b],
                                     sem_g.at[b])

    def _scat(b):
        return pltpu.make_async_copy(rows.at[b], acc.at[didx[b]],
                                     sem_s.at[b])

    for b in range(NBUF):
        for d in _ld_idx(b, b):
            d.start()

    @pl.when(c == 0)
    def _():

        @pl.when(s < NS - 1)
        def _():
            pltpu.sync_copy(y_hbm.at[pl.ds(s * RPT, RPT)],
                            acc.at[pl.ds(s * RPT, RPT)])

        @pl.when(s == NS - 1)
        def _():
            pltpu.sync_copy(y_hbm.at[pl.ds((NS - 1) * RPT, N - (NS - 1) * RPT)],
                            acc.at[pl.ds((NS - 1) * RPT, N - (NS - 1) * RPT)])

            @pl.loop(0, (NP - N) // ZCH)
            def _zpad(k):
                pltpu.sync_copy(zbuf, acc.at[pl.ds(N + k * ZCH, ZCH)])

    @pl.when(c == 1)
    def _():

        @pl.loop(0, RPT // ZCH)
        def _zero_loop(k):
            pltpu.sync_copy(zbuf, acc.at[pl.ds(s * RPT + k * ZCH, ZCH)])

    plsc.subcore_barrier()

    for b in range(NBUF):
        for d in _ld_idx(b, b):
            d.wait()
        _gath(b).start()

    @pl.loop(0, NGA)
    def _edge_loop(g):
        for b in range(NBUF):
            _gath(b).wait()
            _scat(b).start(add=True)
        for b in range(NBUF):

            @pl.when(g < NGA - 1)
            def _():
                _scat(b).wait()
                nxt = (g + 1) * NBUF + b
                dsc = _ld_idx(nxt, b)
                for d in dsc:
                    d.start()
                for d in dsc:
                    d.wait()
                _gath(b).start()

    for b in range(NBUF):
        _scat(b).wait()

    plsc.subcore_barrier()
    pltpu.sync_copy(acc.at[pl.ds(s * RPT, RPT)],
                    out_hbm.at[c, pl.ds(s * RPT, RPT)])


def _leaky(v):
    return jnp.where(v >= 0, v, 0.01 * v)


def _layer_norm(v, g, b):
    mu = jnp.mean(v, axis=-1, keepdims=True)
    var = jnp.mean((v - mu) ** 2, axis=-1, keepdims=True)
    return (v - mu) * lax.rsqrt(var + 1e-5) * g + b


def _mm_body(x_ref, w_ref, o_ref):
    o_ref[...] = jnp.dot(x_ref[...], w_ref[...], preferred_element_type=_f32)


def _dis_col(dpc_ref):
    deg_row = dpc_ref[0:1, :] + dpc_ref[1:2, :]
    deg_col = jnp.transpose(deg_row)[:N, :]
    return lax.rsqrt(deg_col + 1.0)


def _scale_body(dpc_ref, xw_ref, y_ref):
    y_ref[...] = _dis_col(dpc_ref) * xw_ref[...]


def _tc2_body(p_ref, dpc_ref, x_ref, w_ref,
              b1_ref, g1_ref, bt1_ref, y2_ref):
    dis = _dis_col(dpc_ref)
    h = dis * (p_ref[0, :N] + p_ref[1, :N]) + b1_ref[...]
    h = _layer_norm(_leaky(h) + x_ref[...], g1_ref[...], bt1_ref[...])
    y2_ref[...] = dis * jnp.dot(h, w_ref[...], preferred_element_type=_f32)


def _tc3_body(p_ref, dpc_ref, x_ref,
              b2_ref, g2_ref, bt2_ref, out_ref):
    dis = _dis_col(dpc_ref)
    h2 = dis * (p_ref[0, :N] + p_ref[1, :N]) + b2_ref[...]
    out_ref[...] = _leaky(
        _layer_norm(h2 + x_ref[...], g2_ref[...], bt2_ref[...]))


def _tc(body, *args, out_shape=None):
    if out_shape is None:
        out_shape = jax.ShapeDtypeStruct((N, D), _f32)
    return pl.pallas_call(body, out_shape=out_shape)(*args)


def kernel(x, edge_index, W1, b1, g1, bt1, W2, b2, g2, bt2):
    src = edge_index[0]
    dst = edge_index[1]
    dpc = _sc_deg(dst.reshape(NW, NB, B))

    xw1 = _tc(_mm_body, x, W1)
    y1 = _tc(_scale_body, dpc, xw1)
    p1 = _sc_agg(y1, src, dst)
    y2 = _tc(_tc2_body, p1, dpc, x, W2,
             b1.reshape(1, D), g1.reshape(1, D), bt1.reshape(1, D))
    p2 = _sc_agg(y2, src, dst)
    return _tc(_tc3_body, p2, dpc, x,
               b2.reshape(1, D), g2.reshape(1, D), bt2.reshape(1, D))
